# baseline (speedup 1.0000x reference)
import functools

import jax
import jax.numpy as jnp
from jax import lax
from jax.experimental import pallas as pl
from jax.experimental.pallas import tpu as pltpu, tpu_sc as plsc

BATCH = 16384
EMBED_K = 32
TROW = 128
RPB = TROW // EMBED_K

_info = plsc.get_sparse_core_info()
_NC, _NS, _L = _info.num_cores, _info.num_subcores, _info.num_lanes
_NW = _NC * _NS
_BPW = BATCH // _NW
_CHUNK = 128
_NCHUNK = _BPW // _CHUNK
_GPC = _CHUNK // _L


def _sc_body(uidx_hbm, iidx_hbm, user_hbm, item_hbm, w1_hbm, w0_hbm,
             y1_hbm, y0_hbm,
             idx_u, idx_i, gidx, uw, iw, w1_v, w0_v, y1_v, y0_v, sem):
    wid = lax.axis_index("s") * _NC + lax.axis_index("c")
    base = wid * _BPW

    pltpu.sync_copy(uidx_hbm.at[wid], idx_u)
    pltpu.sync_copy(iidx_hbm.at[wid], idx_i)
    pltpu.sync_copy(w1_hbm, w1_v)
    pltpu.sync_copy(w0_hbm, w0_v)

    iota = lax.broadcasted_iota(jnp.int32, (_L,), 0)

    def start_chunk(j):
        for t in range(_CHUNK // _L):
            sl = pl.ds(t * _L, _L)
            gidx[0, sl] = lax.shift_right_logical(idx_u[j, sl], RPB // 2)
            gidx[1, sl] = lax.shift_right_logical(idx_i[j, sl], RPB // 2)
        b = j % 2
        return (
            pltpu.async_copy(user_hbm.at[gidx.at[0]], uw.at[b], sem),
            pltpu.async_copy(item_hbm.at[gidx.at[1]], iw.at[b], sem),
        )

    w1_regs = [w1_v[pl.ds(t * _L, _L)] for t in range(2 * EMBED_K // _L)]
    w0_regs = [w0_v[pl.ds(t * _L, _L)] for t in range(2 * EMBED_K // _L)]

    def _w(regs, k):
        return regs[k // _L][k % _L]

    inflight = start_chunk(0)
    for j in range(_NCHUNK):
        for c in inflight:
            c.wait()
        if j + 1 < _NCHUNK:
            inflight = start_chunk(j + 1)
        b = j % 2

        def group(g, carry, j=j, b=b):
            rows = g * _L + iota
            qu = (idx_u[j, pl.ds(g * _L, _L)] & (RPB - 1)) * EMBED_K
            qi = (idx_i[j, pl.ds(g * _L, _L)] & (RPB - 1)) * EMBED_K
            acc1 = jnp.zeros((_L,), jnp.float32)
            acc0 = jnp.zeros((_L,), jnp.float32)
            for k in range(EMBED_K):
                uv = plsc.load_gather(uw.at[b], [rows, qu + k])
                iv = plsc.load_gather(iw.at[b], [rows, qi + k])
                acc1 = acc1 + uv * _w(w1_regs, k) + iv * _w(w1_regs, EMBED_K + k)
                acc0 = acc0 + uv * _w(w0_regs, k) + iv * _w(w0_regs, EMBED_K + k)
            off = (j * _GPC + g) * _L
            y1_v[pl.ds(off, _L)] = acc1
            y0_v[pl.ds(off, _L)] = acc0
            return carry

        lax.fori_loop(0, _GPC, group, 0, unroll=False)

    pltpu.sync_copy(y1_v, y1_hbm.at[pl.ds(base, _BPW)])
    pltpu.sync_copy(y0_v, y0_hbm.at[pl.ds(base, _BPW)])


@jax.jit
def _sc_call(uidx, iidx, user_table, item_table, w1, w0):
    mesh = plsc.VectorSubcoreMesh(core_axis_name="c", subcore_axis_name="s")
    f = functools.partial(
        pl.kernel,
        mesh=mesh,
        compiler_params=pltpu.CompilerParams(needs_layout_passes=False,
                                             use_tc_tiling_on_sc=False),
        out_type=(
            jax.ShapeDtypeStruct((BATCH,), jnp.float32),
            jax.ShapeDtypeStruct((BATCH,), jnp.float32),
        ),
        scratch_types=[
            pltpu.VMEM((_NCHUNK, _CHUNK), jnp.int32),
            pltpu.VMEM((_NCHUNK, _CHUNK), jnp.int32),
            pltpu.VMEM((2, _CHUNK), jnp.int32),
            pltpu.VMEM((2, _CHUNK, TROW), jnp.float32),
            pltpu.VMEM((2, _CHUNK, TROW), jnp.float32),
            pltpu.VMEM((2 * EMBED_K,), jnp.float32),
            pltpu.VMEM((2 * EMBED_K,), jnp.float32),
            pltpu.VMEM((_BPW,), jnp.float32),
            pltpu.VMEM((_BPW,), jnp.float32),
            pltpu.SemaphoreType.DMA,
        ],
    )(_sc_body)
    return f(uidx, iidx, user_table, item_table, w1, w0)


def kernel(x, user_table, item_table, W1, W0):
    x = x.astype(jnp.int32)
    uidx = x[:, 0].reshape(_NW, _NCHUNK, _CHUNK)
    iidx = x[:, 1].reshape(_NW, _NCHUNK, _CHUNK)
    ut = user_table.reshape(-1, TROW)
    it = item_table.reshape(-1, TROW)
    w1 = W1.reshape(2 * EMBED_K)
    w0 = W0.reshape(2 * EMBED_K)
    y1, y0 = _sc_call(uidx, iidx, ut, it, w1, w0)
    return (y1.reshape(BATCH, 1), y0.reshape(BATCH, 1))
